# baseline (speedup 1.0000x reference)
import functools

import jax
import jax.numpy as jnp
from jax import lax
from jax.experimental import pallas as pl
from jax.experimental.pallas import tpu as pltpu
from jax.experimental.pallas import tpu_sc as plsc

L = 16
NC = 2
NS = 16
NW = NC * NS
NROW = 200
NCOL = 16384
NCLASS = 26
COLS_W = NCOL // NW
CCHUNK = 128
NCHUNK = COLS_W // CCHUNK
NVEC = CCHUNK // L
NACC = 4

_mesh = plsc.VectorSubcoreMesh(core_axis_name="c", subcore_axis_name="s")


@functools.partial(
    pl.kernel,
    mesh=_mesh,
    out_type=jax.ShapeDtypeStruct((NW, L), jnp.float32),
    compiler_params=pltpu.CompilerParams(needs_layout_passes=False),
    scratch_types=[
        pltpu.VMEM((2 * L,), jnp.float32),
        pltpu.VMEM((2, NROW, CCHUNK), jnp.float32),
        pltpu.VMEM((2, NROW, CCHUNK), jnp.int32),
        pltpu.VMEM((L,), jnp.float32),
        pltpu.SemaphoreType.DMA,
        pltpu.SemaphoreType.DMA,
    ],
)
def _wae_sc(x_hbm, t_hbm, cw_hbm, out_hbm, table_v, xb, tb, pv, sem0, sem1):
    wid = lax.axis_index("s") * NC + lax.axis_index("c")
    base = wid * COLS_W
    sems = (sem0, sem1)

    pltpu.sync_copy(cw_hbm, table_v.at[pl.ds(0, NCLASS)])

    def start(c):
        b = c % 2
        src = pl.ds(base + c * CCHUNK, CCHUNK)
        return (
            pltpu.async_copy(x_hbm.at[:, src], xb.at[b], sems[b]),
            pltpu.async_copy(t_hbm.at[:, src], tb.at[b], sems[b]),
        )

    inflight = start(0)
    accs = (jnp.zeros((L,), jnp.float32),) * NACC
    for c in range(NCHUNK):
        for h in inflight:
            h.wait()
        if c + 1 < NCHUNK:
            inflight = start(c + 1)
        b = c % 2

        def body(r, a):
            a = list(a)
            for j in range(NVEC):
                s = pl.ds(j * L, L)
                xv = xb[b, r, s]
                tv = tb[b, r, s]
                w = plsc.load_gather(table_v, [tv])
                wd = w * jnp.abs(xv - tv.astype(jnp.float32))
                a[j % NACC] = a[j % NACC] + wd
            return tuple(a)

        accs = plsc.parallel_loop(0, NROW, 1, unroll=2, carry=accs)(body)

    pv[...] = accs[0] + accs[1] + accs[2] + accs[3]
    pltpu.sync_copy(pv, out_hbm.at[wid])


def kernel(inputs, targets, class_weights):
    partials = _wae_sc(inputs.T, targets.astype(jnp.int32).T, class_weights)
    m = jnp.sum(class_weights > 0).astype(jnp.float32)
    C = jnp.where(m > 0, 1.0 / m, 1.0)
    return C * jnp.sum(partials)


# speedup vs baseline: 1.0588x; 1.0588x over previous
import functools

import jax
import jax.numpy as jnp
from jax import lax
from jax.experimental import pallas as pl
from jax.experimental.pallas import tpu as pltpu
from jax.experimental.pallas import tpu_sc as plsc

L = 16
NC = 2
NS = 16
NW = NC * NS
NROW = 200
NCOL = 16384
NCLASS = 26

SC_COLS = 12288
TC_COLS = NCOL - SC_COLS
COLS_W = SC_COLS // NW
CCHUNK = 128
NCHUNK = COLS_W // CCHUNK
NVEC = CCHUNK // L
NACC = 4
TC_BLOCK = 512
TC_GRID = TC_COLS // TC_BLOCK

_mesh = plsc.VectorSubcoreMesh(core_axis_name="c", subcore_axis_name="s")


@functools.partial(
    pl.kernel,
    mesh=_mesh,
    out_type=jax.ShapeDtypeStruct((NW, L), jnp.float32),
    compiler_params=pltpu.CompilerParams(needs_layout_passes=False),
    scratch_types=[
        pltpu.VMEM((2 * L,), jnp.float32),
        pltpu.VMEM((2, NROW, CCHUNK), jnp.float32),
        pltpu.VMEM((2, NROW, CCHUNK), jnp.int32),
        pltpu.VMEM((L,), jnp.float32),
        pltpu.SemaphoreType.DMA,
        pltpu.SemaphoreType.DMA,
    ],
)
def _wae_sc(x_hbm, t_hbm, cw_hbm, out_hbm, table_v, xb, tb, pv, sem0, sem1):
    wid = lax.axis_index("s") * NC + lax.axis_index("c")
    base = wid * COLS_W
    sems = (sem0, sem1)

    pltpu.sync_copy(cw_hbm, table_v.at[pl.ds(0, NCLASS)])

    def start(c):
        b = c % 2
        src = pl.ds(base + c * CCHUNK, CCHUNK)
        return (
            pltpu.async_copy(x_hbm.at[:, src], xb.at[b], sems[b]),
            pltpu.async_copy(t_hbm.at[:, src], tb.at[b], sems[b]),
        )

    inflight = start(0)
    accs = (jnp.zeros((L,), jnp.float32),) * NACC
    for c in range(NCHUNK):
        for h in inflight:
            h.wait()
        if c + 1 < NCHUNK:
            inflight = start(c + 1)
        b = c % 2

        def body(r, a):
            a = list(a)
            for j in range(NVEC):
                s = pl.ds(j * L, L)
                xv = xb[b, r, s]
                tv = tb[b, r, s]
                w = plsc.load_gather(table_v, [tv])
                wd = w * jnp.abs(xv - tv.astype(jnp.float32))
                a[j % NACC] = a[j % NACC] + wd
            return tuple(a)

        accs = plsc.parallel_loop(0, NROW, 1, unroll=1, carry=accs)(body)

    pv[...] = accs[0] + accs[1] + accs[2] + accs[3]
    pltpu.sync_copy(pv, out_hbm.at[wid])


def _wae_tc_body(x_ref, t_ref, cw_ref, out_ref):
    j = pl.program_id(0)
    x = x_ref[...]
    t = t_ref[...]
    d = jnp.abs(x - t.astype(jnp.float32))
    w = jnp.zeros_like(d)
    for k in range(NCLASS):
        w = jnp.where(t == k, cw_ref[k], w)
    s = jnp.sum(w * d)

    @pl.when(j == 0)
    def _():
        out_ref[0, 0] = 0.0

    out_ref[0, 0] += s


_wae_tc = pl.pallas_call(
    _wae_tc_body,
    grid=(TC_GRID,),
    in_specs=[
        pl.BlockSpec((NROW, TC_BLOCK), lambda j: (0, SC_COLS // TC_BLOCK + j)),
        pl.BlockSpec((NROW, TC_BLOCK), lambda j: (0, SC_COLS // TC_BLOCK + j)),
        pl.BlockSpec(memory_space=pltpu.SMEM),
    ],
    out_specs=pl.BlockSpec(memory_space=pltpu.SMEM),
    out_shape=jax.ShapeDtypeStruct((1, 1), jnp.float32),
)


def kernel(inputs, targets, class_weights):
    xt = inputs.T
    tt = targets.astype(jnp.int32).T
    sc_partials = _wae_sc(xt, tt, class_weights)
    tc_partial = _wae_tc(xt, tt, class_weights)
    m = jnp.sum(class_weights > 0).astype(jnp.float32)
    C = jnp.where(m > 0, 1.0 / m, 1.0)
    return C * (jnp.sum(sc_partials) + tc_partial[0, 0])


# speedup vs baseline: 1.0649x; 1.0057x over previous
import functools

import jax
import jax.numpy as jnp
from jax import lax
from jax.experimental import pallas as pl
from jax.experimental.pallas import tpu as pltpu
from jax.experimental.pallas import tpu_sc as plsc

L = 16
NC = 2
NS = 16
NW = NC * NS
NROW = 200
NCOL = 16384
NCLASS = 26

SC_COLS = 12288
TC_COLS = NCOL - SC_COLS
COLS_W = SC_COLS // NW
CCHUNK = 128
NCHUNK = COLS_W // CCHUNK
NVEC = CCHUNK // L
NACC = 4
TC_BLOCK = 512
TC_GRID = TC_COLS // TC_BLOCK

_mesh = plsc.VectorSubcoreMesh(core_axis_name="c", subcore_axis_name="s")


@functools.partial(
    pl.kernel,
    mesh=_mesh,
    out_type=jax.ShapeDtypeStruct((NW, L), jnp.float32),
    compiler_params=pltpu.CompilerParams(needs_layout_passes=False),
    scratch_types=[
        pltpu.VMEM((2 * L,), jnp.float32),
        pltpu.VMEM((2, NROW, CCHUNK), jnp.float32),
        pltpu.VMEM((2, NROW, CCHUNK), jnp.int32),
        pltpu.VMEM((L,), jnp.float32),
        pltpu.SemaphoreType.DMA,
        pltpu.SemaphoreType.DMA,
    ],
)
def _wae_sc(x_hbm, t_hbm, cw_hbm, out_hbm, table_v, xb, tb, pv, sem0, sem1):
    wid = lax.axis_index("s") * NC + lax.axis_index("c")
    base = wid * COLS_W
    sems = (sem0, sem1)

    pltpu.sync_copy(cw_hbm, table_v.at[pl.ds(0, NCLASS)])

    def start(c):
        b = c % 2
        src = pl.ds(base + c * CCHUNK, CCHUNK)
        return (
            pltpu.async_copy(x_hbm.at[:, src], xb.at[b], sems[b]),
            pltpu.async_copy(t_hbm.at[:, src], tb.at[b], sems[b]),
        )

    inflight = start(0)
    accs = (jnp.zeros((L,), jnp.float32),) * NACC
    for c in range(NCHUNK):
        for h in inflight:
            h.wait()
        if c + 1 < NCHUNK:
            inflight = start(c + 1)
        b = c % 2

        def body(r, a):
            a = list(a)
            for j in range(NVEC):
                s = pl.ds(j * L, L)
                xv = xb[b, r, s]
                tv = tb[b, r, s]
                w = plsc.load_gather(table_v, [tv])
                wd = w * jnp.abs(xv - tv.astype(jnp.float32))
                a[j % NACC] = a[j % NACC] + wd
            return tuple(a)

        accs = plsc.parallel_loop(0, NROW, 1, unroll=1, carry=accs)(body)

    pv[...] = accs[0] + accs[1] + accs[2] + accs[3]
    pltpu.sync_copy(pv, out_hbm.at[wid])


def _wae_tc_body(x_ref, t_ref, cw_ref, out_ref):
    j = pl.program_id(0)
    x = x_ref[...]
    t = t_ref[...]
    d = jnp.abs(x - t.astype(jnp.float32))
    cw = cw_ref[...]
    w = jnp.zeros_like(d)
    for k in range(NCLASS):
        w = jnp.where(t == k, cw[k], w)
    s = jnp.sum(w * d)

    @pl.when(j == 0)
    def _():
        out_ref[0, 0] = 0.0

    out_ref[0, 0] += s


_wae_tc = pl.pallas_call(
    _wae_tc_body,
    grid=(TC_GRID,),
    in_specs=[
        pl.BlockSpec((NROW, TC_BLOCK), lambda j: (0, SC_COLS // TC_BLOCK + j)),
        pl.BlockSpec((NROW, TC_BLOCK), lambda j: (0, SC_COLS // TC_BLOCK + j)),
        pl.BlockSpec((NCLASS,), lambda j: (0,)),
    ],
    out_specs=pl.BlockSpec(memory_space=pltpu.SMEM),
    out_shape=jax.ShapeDtypeStruct((1, 1), jnp.float32),
)


def kernel(inputs, targets, class_weights):
    xt = inputs.T
    tt = targets.astype(jnp.int32).T
    sc_partials = _wae_sc(xt, tt, class_weights)
    tc_partial = _wae_tc(xt, tt, class_weights)
    m = jnp.sum(class_weights > 0).astype(jnp.float32)
    C = jnp.where(m > 0, 1.0 / m, 1.0)
    return C * (jnp.sum(sc_partials) + tc_partial[0, 0])
